# Initial kernel scaffold; baseline (speedup 1.0000x reference)
import jax
import jax.numpy as jnp
from jax.experimental import pallas as pl


def kernel(x, edge_index, edge_attr, batch_ids, W1, b1, W2, b2):
    raise NotImplementedError("write your pallas kernel here")


# speedup vs baseline: 2.5937x; 2.5937x over previous
import functools

import jax
import jax.numpy as jnp
from jax import lax
from jax.experimental import pallas as pl
from jax.experimental.pallas import tpu as pltpu
from jax.experimental.pallas import tpu_sc as plsc

N_NODES = 10000
N_EDGES = 320000
D_FEAT = 128
HIDDEN = 64
D_EDGE = 16
N_CLASSES = 2

SC_CORES = 2
SC_SUBCORES = 16
NW = SC_CORES * SC_SUBCORES
EPW = N_EDGES // NW
CH = 80
N_CHUNKS = EPW // CH


def _project(x, w1s, w1d):
    def body(x_ref, ws_ref, wd_ref, os_ref, od_ref):
        xb = x_ref[...]
        os_ref[...] = jnp.dot(xb, ws_ref[...], preferred_element_type=jnp.float32)
        od_ref[...] = jnp.dot(xb, wd_ref[...], preferred_element_type=jnp.float32)

    rb = 1000
    return pl.pallas_call(
        body,
        grid=(N_NODES // rb,),
        in_specs=[
            pl.BlockSpec((rb, D_FEAT), lambda i: (i, 0)),
            pl.BlockSpec((D_FEAT, HIDDEN), lambda i: (0, 0)),
            pl.BlockSpec((D_FEAT, HIDDEN), lambda i: (0, 0)),
        ],
        out_specs=[
            pl.BlockSpec((rb, HIDDEN), lambda i: (i, 0)),
            pl.BlockSpec((rb, HIDDEN), lambda i: (i, 0)),
        ],
        out_shape=[
            jax.ShapeDtypeStruct((N_NODES, HIDDEN), jnp.float32),
            jax.ShapeDtypeStruct((N_NODES, HIDDEN), jnp.float32),
        ],
    )(x, w1s, w1d)


def _gather_sum(xs, xd, src3, dst3):
    mesh = plsc.VectorSubcoreMesh(core_axis_name="c", subcore_axis_name="s")

    @functools.partial(
        pl.kernel,
        out_type=jax.ShapeDtypeStruct((N_EDGES, HIDDEN), jnp.float32),
        mesh=mesh,
        scratch_types=[
            pltpu.VMEM((N_CHUNKS, CH), jnp.int32),
            pltpu.VMEM((N_CHUNKS, CH), jnp.int32),
            pltpu.VMEM((CH, HIDDEN), jnp.float32),
        ],
        compiler_params=pltpu.CompilerParams(use_tc_tiling_on_sc=False),
    )
    def k(xs_hbm, xd_hbm, src_hbm, dst_hbm, g_hbm, src_v, dst_v, gbuf):
        wid = lax.axis_index("s") * SC_CORES + lax.axis_index("c")
        base = wid * EPW
        pltpu.sync_copy(src_hbm.at[wid], src_v)
        pltpu.sync_copy(dst_hbm.at[wid], dst_v)

        def body(j, carry):
            pltpu.sync_copy(xs_hbm.at[src_v.at[j]], gbuf)
            pltpu.sync_copy(xd_hbm.at[dst_v.at[j]], gbuf, add=True)
            pltpu.sync_copy(gbuf, g_hbm.at[pl.ds(base + j * CH, CH)])
            return carry

        lax.fori_loop(0, N_CHUNKS, body, 0)

    return k(xs, xd, src3, dst3)


def _mlp(g, edge_attr, w1e, b1, w2, b2):
    def body(g_ref, ea_ref, w1e_ref, b1_ref, w2_ref, b2_ref, o_ref):
        h = g_ref[...] + jnp.dot(ea_ref[...], w1e_ref[...],
                                 preferred_element_type=jnp.float32)
        h = jnp.maximum(h + b1_ref[0:1, :], 0.0)
        o_ref[...] = jnp.dot(h, w2_ref[...],
                             preferred_element_type=jnp.float32) + b2_ref[0:1, :]

    cb = 5000
    b1p = jnp.tile(b1.reshape(1, HIDDEN), (8, 1))
    b2p = jnp.tile(b2.reshape(1, N_CLASSES), (8, 1))
    return pl.pallas_call(
        body,
        grid=(N_EDGES // cb,),
        in_specs=[
            pl.BlockSpec((cb, HIDDEN), lambda i: (i, 0)),
            pl.BlockSpec((cb, D_EDGE), lambda i: (i, 0)),
            pl.BlockSpec((D_EDGE, HIDDEN), lambda i: (0, 0)),
            pl.BlockSpec((8, HIDDEN), lambda i: (0, 0)),
            pl.BlockSpec((HIDDEN, N_CLASSES), lambda i: (0, 0)),
            pl.BlockSpec((8, N_CLASSES), lambda i: (0, 0)),
        ],
        out_specs=pl.BlockSpec((cb, N_CLASSES), lambda i: (i, 0)),
        out_shape=jax.ShapeDtypeStruct((N_EDGES, N_CLASSES), jnp.float32),
    )(g, edge_attr, w1e, b1p, w2, b2p)


def kernel(x, edge_index, edge_attr, batch_ids, W1, b1, W2, b2):
    del batch_ids
    src = edge_index[0].astype(jnp.int32).reshape(NW, N_CHUNKS, CH)
    dst = edge_index[1].astype(jnp.int32).reshape(NW, N_CHUNKS, CH)
    w1s = W1[0:D_FEAT]
    w1d = W1[D_FEAT:2 * D_FEAT]
    w1e = W1[2 * D_FEAT:]
    xs, xd = _project(x, w1s, w1d)
    g = _gather_sum(xs, xd, src, dst)
    return _mlp(g, edge_attr, w1e, b1, W2, b2)
